# baseline (speedup 1.0000x reference)
import functools

import jax
import jax.numpy as jnp
from jax import lax
from jax.experimental import pallas as pl
from jax.experimental.pallas import tpu as pltpu
from jax.experimental.pallas import tpu_sc as plsc

N = 10000
E = 320000
H = 128

NC = 2
NS = 16
EP = 337920
EPT = EP // NS
K = 80
ST = 66
CPS = 4
NBUF = 4
DW = 4
NP = 10112
RPT = NP // NS

F32 = jnp.float32


def _mesh():
    return plsc.VectorSubcoreMesh(
        core_axis_name="c", subcore_axis_name="s",
        num_cores=NC, num_subcores=NS)


def _sc_deg(edg):
    @functools.partial(
        pl.kernel,
        out_type=jax.ShapeDtypeStruct((NC, NP, 16), F32),
        mesh=_mesh(),
        scratch_types=[
            pltpu.VMEM((2, CPS, K), jnp.int32),
            pltpu.VMEM((K, 16), F32),
            pltpu.VMEM_SHARED((NP, 16), F32),
            pltpu.SemaphoreType.DMA,
            pltpu.SemaphoreType.DMA,
        ],
    )
    def k(edg_hbm, out_hbm, slab, ones_v, acc, sem, lsem):
        c = lax.axis_index("c")
        s = lax.axis_index("s")
        base = s * RPT

        def _fill(val):
            def _f(i, carry):
                ones_v[i, :] = jnp.full((16,), val, F32)
                return carry
            lax.fori_loop(0, K, _f, 0)

        _fill(0.0)
        for j in range(RPT // K):
            pltpu.sync_copy(ones_v, acc.at[pl.ds(base + j * K, K)])
        pltpu.sync_copy(ones_v.at[pl.ds(0, RPT % K)],
                        acc.at[pl.ds(base + (RPT // K) * K, RPT % K)])
        plsc.subcore_barrier()
        _fill(1.0)

        def _fire(p, r):
            pltpu.async_copy(ones_v, acc.at[slab.at[p, r]], sem, add=True)

        def _drain(p, r):
            pltpu.make_async_copy(ones_v, acc.at[slab.at[p, r]], sem).wait()

        pltpu.sync_copy(edg_hbm.at[c, s, 0, 1], slab.at[0])
        for b in range(ST):
            p = b % 2
            if b + 1 < ST:
                pltpu.async_copy(edg_hbm.at[c, s, b + 1, 1],
                                 slab.at[1 - p], lsem)
            for q in range(DW):
                _fire(p, q)

            def _step(j, carry):
                _drain(p, j)
                _fire(p, j + DW)
                return carry
            lax.fori_loop(0, CPS - DW, _step, 0)
            for q in range(CPS - DW, CPS):
                _drain(p, q)
            if b + 1 < ST:
                pltpu.make_async_copy(edg_hbm.at[c, s, b + 1, 1],
                                      slab.at[1 - p], lsem).wait()
        plsc.subcore_barrier()

        pltpu.sync_copy(acc.at[pl.ds(base, RPT)],
                        out_hbm.at[c, pl.ds(base, RPT)])

    return k(edg)


def _sc_scatter(xs, edg):
    @functools.partial(
        pl.kernel,
        out_type=jax.ShapeDtypeStruct((NC, NP, H), F32),
        mesh=_mesh(),
        scratch_types=[
            pltpu.VMEM((2, 2, CPS, K), jnp.int32),
            pltpu.VMEM((NBUF, K, H), F32),
            pltpu.VMEM_SHARED((NP, H), F32),
            pltpu.SemaphoreType.DMA,
            pltpu.SemaphoreType.DMA,
            pltpu.SemaphoreType.DMA,
            pltpu.SemaphoreType.DMA,
            pltpu.SemaphoreType.DMA,
            pltpu.SemaphoreType.DMA,
            pltpu.SemaphoreType.DMA,
            pltpu.SemaphoreType.DMA,
            pltpu.SemaphoreType.DMA,
        ],
    )
    def k(xs_hbm, edg_hbm, out_hbm,
          slab, rows, acc,
          g0, g1, g2, g3, s0, s1, s2, s3, lsem):
        gs = (g0, g1, g2, g3)
        ss = (s0, s1, s2, s3)
        c = lax.axis_index("c")
        s = lax.axis_index("s")
        base = s * RPT

        def _gwait(q, p, r):
            pltpu.make_async_copy(
                xs_hbm.at[slab.at[p, 0, r]], rows.at[q], gs[q]).wait()

        def _swait(q, p, r):
            pltpu.make_async_copy(
                rows.at[q], acc.at[slab.at[p, 1, r]], ss[q]).wait()

        def _zrow(i, carry):
            for q in range(NBUF):
                for j in range(H // 16):
                    rows[q, i, pl.ds(j * 16, 16)] = jnp.zeros((16,), F32)
            return carry
        lax.fori_loop(0, K, _zrow, 0)
        for j in range(RPT // K):
            pltpu.sync_copy(rows.at[j % NBUF],
                            acc.at[pl.ds(base + j * K, K)])
        pltpu.sync_copy(rows.at[NBUF - 1, pl.ds(0, RPT % K)],
                        acc.at[pl.ds(base + (RPT // K) * K, RPT % K)])
        plsc.subcore_barrier()

        pltpu.sync_copy(edg_hbm.at[c, s, 0], slab.at[0])
        for q in range(NBUF):
            pltpu.async_copy(xs_hbm.at[slab.at[0, 0, q]], rows.at[q],
                             gs[q])
        for b in range(ST):
            p = b % 2
            if b + 1 < ST:
                pltpu.async_copy(edg_hbm.at[c, s, b + 1],
                                 slab.at[1 - p], lsem)

            def _round(j, carry):
                for q in range(NBUF):
                    r = j * NBUF + q
                    _gwait(q, p, r)
                    pltpu.async_copy(rows.at[q], acc.at[slab.at[p, 1, r]],
                                     ss[q], add=True)
                for q in range(NBUF):
                    r = j * NBUF + q
                    _swait(q, p, r)
                    pltpu.async_copy(xs_hbm.at[slab.at[p, 0, r + NBUF]],
                                     rows.at[q], gs[q])
                return carry
            lax.fori_loop(0, CPS // NBUF - 1, _round, 0)

            for q in range(NBUF):
                r = CPS - NBUF + q
                _gwait(q, p, r)
                pltpu.async_copy(rows.at[q], acc.at[slab.at[p, 1, r]],
                                 ss[q], add=True)
            if b + 1 < ST:
                pltpu.make_async_copy(edg_hbm.at[c, s, b + 1],
                                      slab.at[1 - p], lsem).wait()
            for q in range(NBUF):
                _swait(q, p, CPS - NBUF + q)
                if b + 1 < ST:
                    pltpu.async_copy(xs_hbm.at[slab.at[1 - p, 0, q]],
                                     rows.at[q], gs[q])
        plsc.subcore_barrier()

        pltpu.sync_copy(acc.at[pl.ds(base, RPT)],
                        out_hbm.at[c, pl.ds(base, RPT)])

    return k(xs, edg)


_R = 1000


def _g_row_spec():
    return pl.BlockSpec((1, _R, H), lambda g, i: (g, i, 0))


def _g_w_spec():
    return pl.BlockSpec((1, H, H), lambda g, i: (g, 0, 0))


def _g_deg_spec():
    return pl.BlockSpec((1, _R, 16), lambda g, i: (g, i, 0))


def _g_b_spec():
    return pl.BlockSpec((1, 1, H), lambda g, i: (g, 0, 0))


def _prep_body(x_ref, wt_ref, dg_ref, o_ref):
    dinv = lax.rsqrt(dg_ref[0, :, 0])[:, None]
    o_ref[0] = dinv * jnp.dot(x_ref[0], wt_ref[0],
                              preferred_element_type=F32)


def _tc_prep(x, wt, degp):
    return pl.pallas_call(
        _prep_body,
        grid=(2, N // _R),
        in_specs=[_g_row_spec(), _g_w_spec(), _g_deg_spec()],
        out_specs=_g_row_spec(),
        out_shape=jax.ShapeDtypeStruct((2, N, H), F32),
    )(x, wt, degp)


def _mid_body(p_ref, dg_ref, b_ref, wt_ref, o_ref):
    dinv = lax.rsqrt(dg_ref[0, :, 0])[:, None]
    h = jnp.maximum(dinv * p_ref[0] + b_ref[0], 0.0)
    o_ref[0] = dinv * jnp.dot(h, wt_ref[0], preferred_element_type=F32)


def _tc_mid(parts, degp, b, wt):
    return pl.pallas_call(
        _mid_body,
        grid=(2, N // _R),
        in_specs=[_g_row_spec(), _g_deg_spec(), _g_b_spec(), _g_w_spec()],
        out_specs=_g_row_spec(),
        out_shape=jax.ShapeDtypeStruct((2, N, H), F32),
    )(parts, degp, b, wt)


def _fin_body(p_ref, dg_ref, b_ref, fwi_ref, fws_ref, fb_ref,
              o_ref):
    dinv_i = lax.rsqrt(dg_ref[0, :, 0])[:, None]
    h_i = jnp.maximum(dinv_i * p_ref[0] + b_ref[0], 0.0)
    dinv_s = lax.rsqrt(dg_ref[1, :, 0])[:, None]
    h_s = jnp.maximum(dinv_s * p_ref[1] + b_ref[1], 0.0)
    o_ref[...] = (jnp.dot(h_i, fwi_ref[...], preferred_element_type=F32)
                  + jnp.dot(h_s, fws_ref[...], preferred_element_type=F32)
                  + fb_ref[...])


def _tc_fin(parts, degp, b, fwi, fws, fb):
    return pl.pallas_call(
        _fin_body,
        grid=(N // _R,),
        in_specs=[
            pl.BlockSpec((2, _R, H), lambda i: (0, i, 0)),
            pl.BlockSpec((2, _R, 16), lambda i: (0, i, 0)),
            pl.BlockSpec((2, 1, H), lambda i: (0, 0, 0)),
            pl.BlockSpec((H, H), lambda i: (0, 0)),
            pl.BlockSpec((H, H), lambda i: (0, 0)),
            pl.BlockSpec((1, H), lambda i: (0, 0)),
        ],
        out_specs=pl.BlockSpec((_R, H), lambda i: (i, 0)),
        out_shape=jax.ShapeDtypeStruct((N, H), F32),
    )(parts, degp, b, fwi, fws, fb)


def kernel(interaction_x, interaction_edge_index,
           similarity_x, similarity_edge_index,
           W_ic1, b_ic1, W_ic2, b_ic2,
           W_sc1, b_sc1, W_sc2, b_sc2, fc_W, fc_b):
    npad_n = EP - E - N
    zpad = jnp.arange(npad_n, dtype=jnp.int32) % N
    npad = N + (jnp.arange(npad_n, dtype=jnp.int32) % (NP - N))
    loop = jnp.arange(N, dtype=jnp.int32)
    shp = (NS, ST, CPS, K)

    def _edges(ei, off):
        s = (jnp.concatenate([ei[0], loop, zpad]) + off).reshape(shp)
        d = jnp.concatenate([ei[1], loop, npad]).reshape(shp)
        return jnp.stack([s, d], axis=2)

    edg = jnp.stack([_edges(interaction_edge_index, 0),
                     _edges(similarity_edge_index, N)])

    x_cat = jnp.stack([interaction_x, similarity_x])
    wt1 = jnp.stack([W_ic1.T, W_sc1.T])
    wt2 = jnp.stack([W_ic2.T, W_sc2.T])
    b1 = jnp.stack([b_ic1.reshape(1, H), b_sc1.reshape(1, H)])
    b2 = jnp.stack([b_ic2.reshape(1, H), b_sc2.reshape(1, H)])
    fwt = fc_W.T
    fwt_i = fwt[:H]
    fwt_s = fwt[H:]
    fb = fc_b.reshape(1, H)

    degp = _sc_deg(edg)
    xs1 = _tc_prep(x_cat, wt1, degp)
    p1 = _sc_scatter(xs1.reshape(2 * N, H), edg)
    xs2 = _tc_mid(p1, degp, b1, wt2)
    p2 = _sc_scatter(xs2.reshape(2 * N, H), edg)
    return _tc_fin(p2, degp, b2, fwt_i, fwt_s, fb)


# speedup vs baseline: 1.1070x; 1.1070x over previous
import functools

import jax
import jax.numpy as jnp
from jax import lax
from jax.experimental import pallas as pl
from jax.experimental.pallas import tpu as pltpu
from jax.experimental.pallas import tpu_sc as plsc

N = 10000
E = 320000
H = 128

NC = 2
NS = 16
EP = 327680
EPT = EP // NS
K = 80
ST = 16
CPS = 16
NBUF = 4
DW = 8
NP = 10112
RPT = NP // NS

F32 = jnp.float32


def _mesh():
    return plsc.VectorSubcoreMesh(
        core_axis_name="c", subcore_axis_name="s",
        num_cores=NC, num_subcores=NS)


def _sc_deg(edg):
    @functools.partial(
        pl.kernel,
        out_type=jax.ShapeDtypeStruct((NC, NP, 16), F32),
        mesh=_mesh(),
        scratch_types=[
            pltpu.VMEM((2, CPS, K), jnp.int32),
            pltpu.VMEM((K, 16), F32),
            pltpu.VMEM_SHARED((NP, 16), F32),
            pltpu.SemaphoreType.DMA,
            pltpu.SemaphoreType.DMA,
        ],
    )
    def k(edg_hbm, out_hbm, slab, ones_v, acc, sem, lsem):
        c = lax.axis_index("c")
        s = lax.axis_index("s")
        base = s * RPT

        def _fill(val):
            def _f(i, carry):
                ones_v[i, :] = jnp.full((16,), val, F32)
                return carry
            lax.fori_loop(0, K, _f, 0)

        _fill(0.0)
        for j in range(RPT // K):
            pltpu.sync_copy(ones_v, acc.at[pl.ds(base + j * K, K)])
        pltpu.sync_copy(ones_v.at[pl.ds(0, RPT % K)],
                        acc.at[pl.ds(base + (RPT // K) * K, RPT % K)])
        plsc.subcore_barrier()
        _fill(1.0)

        def _fire(p, r):
            pltpu.async_copy(ones_v, acc.at[slab.at[p, r]], sem, add=True)

        def _drain(p, r):
            pltpu.make_async_copy(ones_v, acc.at[slab.at[p, r]], sem).wait()

        pltpu.sync_copy(edg_hbm.at[c, s, 0, 1], slab.at[0])
        for b in range(ST):
            p = b % 2
            if b + 1 < ST:
                pltpu.async_copy(edg_hbm.at[c, s, b + 1, 1],
                                 slab.at[1 - p], lsem)
            for q in range(DW):
                _fire(p, q)

            def _step(j, carry):
                _drain(p, j)
                _fire(p, j + DW)
                return carry
            lax.fori_loop(0, CPS - DW, _step, 0)
            for q in range(CPS - DW, CPS):
                _drain(p, q)
            if b + 1 < ST:
                pltpu.make_async_copy(edg_hbm.at[c, s, b + 1, 1],
                                      slab.at[1 - p], lsem).wait()
        plsc.subcore_barrier()

        pltpu.sync_copy(acc.at[pl.ds(base, RPT)],
                        out_hbm.at[c, pl.ds(base, RPT)])

    return k(edg)


def _sc_scatter(xs, edg):
    @functools.partial(
        pl.kernel,
        out_type=jax.ShapeDtypeStruct((NC, NP, H), F32),
        mesh=_mesh(),
        scratch_types=[
            pltpu.VMEM((2, 2, CPS, K), jnp.int32),
            pltpu.VMEM((NBUF, K, H), F32),
            pltpu.VMEM_SHARED((NP, H), F32),
            pltpu.SemaphoreType.DMA,
            pltpu.SemaphoreType.DMA,
            pltpu.SemaphoreType.DMA,
            pltpu.SemaphoreType.DMA,
            pltpu.SemaphoreType.DMA,
            pltpu.SemaphoreType.DMA,
            pltpu.SemaphoreType.DMA,
            pltpu.SemaphoreType.DMA,
            pltpu.SemaphoreType.DMA,
        ],
    )
    def k(xs_hbm, edg_hbm, out_hbm,
          slab, rows, acc,
          g0, g1, g2, g3, s0, s1, s2, s3, lsem):
        gs = (g0, g1, g2, g3)
        ss = (s0, s1, s2, s3)
        c = lax.axis_index("c")
        s = lax.axis_index("s")
        base = s * RPT

        def _gwait(q, p, r):
            pltpu.make_async_copy(
                xs_hbm.at[slab.at[p, 0, r]], rows.at[q], gs[q]).wait()

        def _swait(q, p, r):
            pltpu.make_async_copy(
                rows.at[q], acc.at[slab.at[p, 1, r]], ss[q]).wait()

        def _zrow(i, carry):
            for q in range(NBUF):
                for j in range(H // 16):
                    rows[q, i, pl.ds(j * 16, 16)] = jnp.zeros((16,), F32)
            return carry
        lax.fori_loop(0, K, _zrow, 0)
        for j in range(RPT // K):
            pltpu.sync_copy(rows.at[j % NBUF],
                            acc.at[pl.ds(base + j * K, K)])
        pltpu.sync_copy(rows.at[NBUF - 1, pl.ds(0, RPT % K)],
                        acc.at[pl.ds(base + (RPT // K) * K, RPT % K)])
        plsc.subcore_barrier()

        pltpu.sync_copy(edg_hbm.at[c, s, 0], slab.at[0])
        for q in range(NBUF):
            pltpu.async_copy(xs_hbm.at[slab.at[0, 0, q]], rows.at[q],
                             gs[q])
        for b in range(ST):
            p = b % 2
            if b + 1 < ST:
                pltpu.async_copy(edg_hbm.at[c, s, b + 1],
                                 slab.at[1 - p], lsem)

            def _round(j, carry):
                for q in range(NBUF):
                    r = j * NBUF + q
                    _gwait(q, p, r)
                    pltpu.async_copy(rows.at[q], acc.at[slab.at[p, 1, r]],
                                     ss[q], add=True)
                for q in range(NBUF):
                    r = j * NBUF + q
                    _swait(q, p, r)
                    pltpu.async_copy(xs_hbm.at[slab.at[p, 0, r + NBUF]],
                                     rows.at[q], gs[q])
                return carry
            lax.fori_loop(0, CPS // NBUF - 1, _round, 0)

            for q in range(NBUF):
                r = CPS - NBUF + q
                _gwait(q, p, r)
                pltpu.async_copy(rows.at[q], acc.at[slab.at[p, 1, r]],
                                 ss[q], add=True)
            if b + 1 < ST:
                pltpu.make_async_copy(edg_hbm.at[c, s, b + 1],
                                      slab.at[1 - p], lsem).wait()
            for q in range(NBUF):
                _swait(q, p, CPS - NBUF + q)
                if b + 1 < ST:
                    pltpu.async_copy(xs_hbm.at[slab.at[1 - p, 0, q]],
                                     rows.at[q], gs[q])
        plsc.subcore_barrier()

        pltpu.sync_copy(acc.at[pl.ds(base, RPT)],
                        out_hbm.at[c, pl.ds(base, RPT)])

    return k(xs, edg)


_R = 1000


def _g_row_spec():
    return pl.BlockSpec((1, _R, H), lambda g, i: (g, i, 0))


def _g_w_spec():
    return pl.BlockSpec((1, H, H), lambda g, i: (g, 0, 0))


def _g_deg_spec():
    return pl.BlockSpec((1, _R, 16), lambda g, i: (g, i, 0))


def _g_b_spec():
    return pl.BlockSpec((1, 1, H), lambda g, i: (g, 0, 0))


def _prep_body(x_ref, wt_ref, dg_ref, o_ref):
    dinv = lax.rsqrt(1.0 + dg_ref[0, :, 0])[:, None]
    o_ref[0] = dinv * jnp.dot(x_ref[0], wt_ref[0],
                              preferred_element_type=F32)


def _tc_prep(x, wt, degp):
    return pl.pallas_call(
        _prep_body,
        grid=(2, N // _R),
        in_specs=[_g_row_spec(), _g_w_spec(), _g_deg_spec()],
        out_specs=_g_row_spec(),
        out_shape=jax.ShapeDtypeStruct((2, N, H), F32),
    )(x, wt, degp)


def _mid_body(p_ref, xs_ref, dg_ref, b_ref, wt_ref, o_ref):
    dinv = lax.rsqrt(1.0 + dg_ref[0, :, 0])[:, None]
    h = jnp.maximum(dinv * (p_ref[0] + xs_ref[0]) + b_ref[0], 0.0)
    o_ref[0] = dinv * jnp.dot(h, wt_ref[0], preferred_element_type=F32)


def _tc_mid(parts, xs, degp, b, wt):
    return pl.pallas_call(
        _mid_body,
        grid=(2, N // _R),
        in_specs=[_g_row_spec(), _g_row_spec(), _g_deg_spec(),
                  _g_b_spec(), _g_w_spec()],
        out_specs=_g_row_spec(),
        out_shape=jax.ShapeDtypeStruct((2, N, H), F32),
    )(parts, xs, degp, b, wt)


def _fin_body(p_ref, xs_ref, dg_ref, b_ref, fwi_ref, fws_ref, fb_ref,
              o_ref):
    dinv_i = lax.rsqrt(1.0 + dg_ref[0, :, 0])[:, None]
    h_i = jnp.maximum(dinv_i * (p_ref[0] + xs_ref[0]) + b_ref[0], 0.0)
    dinv_s = lax.rsqrt(1.0 + dg_ref[1, :, 0])[:, None]
    h_s = jnp.maximum(dinv_s * (p_ref[1] + xs_ref[1]) + b_ref[1], 0.0)
    o_ref[...] = (jnp.dot(h_i, fwi_ref[...], preferred_element_type=F32)
                  + jnp.dot(h_s, fws_ref[...], preferred_element_type=F32)
                  + fb_ref[...])


def _tc_fin(parts, xs, degp, b, fwi, fws, fb):
    return pl.pallas_call(
        _fin_body,
        grid=(N // _R,),
        in_specs=[
            pl.BlockSpec((2, _R, H), lambda i: (0, i, 0)),
            pl.BlockSpec((2, _R, H), lambda i: (0, i, 0)),
            pl.BlockSpec((2, _R, 16), lambda i: (0, i, 0)),
            pl.BlockSpec((2, 1, H), lambda i: (0, 0, 0)),
            pl.BlockSpec((H, H), lambda i: (0, 0)),
            pl.BlockSpec((H, H), lambda i: (0, 0)),
            pl.BlockSpec((1, H), lambda i: (0, 0)),
        ],
        out_specs=pl.BlockSpec((_R, H), lambda i: (i, 0)),
        out_shape=jax.ShapeDtypeStruct((N, H), F32),
    )(parts, xs, degp, b, fwi, fws, fb)


def kernel(interaction_x, interaction_edge_index,
           similarity_x, similarity_edge_index,
           W_ic1, b_ic1, W_ic2, b_ic2,
           W_sc1, b_sc1, W_sc2, b_sc2, fc_W, fc_b):
    zpad = jnp.arange(EP - E, dtype=jnp.int32) % N
    npad = N + (jnp.arange(EP - E, dtype=jnp.int32) % (NP - N))
    shp = (NS, ST, CPS, K)

    def _edges(ei, off):
        s = (jnp.concatenate([ei[0], zpad]) + off).reshape(shp)
        d = jnp.concatenate([ei[1], npad]).reshape(shp)
        return jnp.stack([s, d], axis=2)

    edg = jnp.stack([_edges(interaction_edge_index, 0),
                     _edges(similarity_edge_index, N)])

    x_cat = jnp.stack([interaction_x, similarity_x])
    wt1 = jnp.stack([W_ic1.T, W_sc1.T])
    wt2 = jnp.stack([W_ic2.T, W_sc2.T])
    b1 = jnp.stack([b_ic1.reshape(1, H), b_sc1.reshape(1, H)])
    b2 = jnp.stack([b_ic2.reshape(1, H), b_sc2.reshape(1, H)])
    fwt = fc_W.T
    fwt_i = fwt[:H]
    fwt_s = fwt[H:]
    fb = fc_b.reshape(1, H)

    degp = _sc_deg(edg)
    xs1 = _tc_prep(x_cat, wt1, degp)
    p1 = _sc_scatter(xs1.reshape(2 * N, H), edg)
    xs2 = _tc_mid(p1, xs1, degp, b1, wt2)
    p2 = _sc_scatter(xs2.reshape(2 * N, H), edg)
    return _tc_fin(p2, xs2, degp, b2, fwt_i, fwt_s, fb)
